# baseline (speedup 1.0000x reference)
import functools

import jax
import jax.numpy as jnp
from jax import lax
from jax.experimental import pallas as pl
from jax.experimental.pallas import tpu as pltpu
from jax.experimental.pallas import tpu_sc as plsc

VOCAB = 100000
EMBED = 128
BATCH = 1024

V_TILE = 4096
GRID_V = -(-VOCAB // V_TILE)


@functools.lru_cache(maxsize=None)
def _make_sc_gather():
    info = plsc.get_sparse_core_info()
    nw = info.num_cores * info.num_subcores
    b_per_w = BATCH // nw
    mesh = plsc.VectorSubcoreMesh(core_axis_name="c", subcore_axis_name="s")

    @functools.partial(
        pl.kernel,
        mesh=mesh,
        out_type=jax.ShapeDtypeStruct((BATCH, EMBED), jnp.float32),
        scratch_types=[
            pltpu.VMEM((b_per_w,), jnp.int32),
            pltpu.VMEM((b_per_w, EMBED), jnp.float32),
            pltpu.SemaphoreType.DMA,
        ],
    )
    def gather(table_hbm, idx_hbm, out_hbm, idx_v, rows_v, sem):
        wid = lax.axis_index("s") * info.num_cores + lax.axis_index("c")
        base = wid * b_per_w
        pltpu.sync_copy(idx_hbm.at[pl.ds(base, b_per_w)], idx_v)
        pltpu.async_copy(table_hbm.at[idx_v], rows_v, sem).wait()
        pltpu.sync_copy(rows_v, out_hbm.at[pl.ds(base, b_per_w)])

    return gather


def _matmul_body(x_ref, w_ref, b_ref, o_ref):
    o_ref[...] = lax.dot_general(
        x_ref[...].astype(jnp.bfloat16), w_ref[...].astype(jnp.bfloat16),
        (((1,), (1,)), ((), ())),
        preferred_element_type=jnp.float32,
    ) + b_ref[...]


def _tc_logits(x, fc_w, fc_b2d):
    return pl.pallas_call(
        _matmul_body,
        grid=(GRID_V,),
        in_specs=[
            pl.BlockSpec((BATCH, EMBED), lambda i: (0, 0)),
            pl.BlockSpec((V_TILE, EMBED), lambda i: (i, 0)),
            pl.BlockSpec((1, V_TILE), lambda i: (0, i)),
        ],
        out_specs=pl.BlockSpec((BATCH, V_TILE), lambda i: (0, i)),
        out_shape=jax.ShapeDtypeStruct((BATCH, VOCAB), jnp.float32),
        compiler_params=pltpu.CompilerParams(
            dimension_semantics=("arbitrary",),
        ),
    )(x, fc_w, fc_b2d)


def kernel(target_word, emb_table, fc_w, fc_b):
    x = jnp.take(emb_table, target_word, axis=0)
    return _tc_logits(x, fc_w, fc_b.reshape(1, VOCAB))


# speedup vs baseline: 1.1393x; 1.1393x over previous
import functools

import jax
import jax.numpy as jnp
from jax import lax
from jax.experimental import pallas as pl
from jax.experimental.pallas import tpu as pltpu
from jax.experimental.pallas import tpu_sc as plsc

VOCAB = 100000
EMBED = 128
BATCH = 1024

V_TILE = 2048
GRID_V = -(-VOCAB // V_TILE)
V_TAIL = VOCAB - (GRID_V - 1) * V_TILE
NQ = 4
ROWS_Q = BATCH // NQ


@functools.lru_cache(maxsize=None)
def _make_sc_gather():
    info = plsc.get_sparse_core_info()
    nw = info.num_cores * info.num_subcores
    b_per_w = BATCH // nw
    mesh = plsc.VectorSubcoreMesh(core_axis_name="c", subcore_axis_name="s")

    @functools.partial(
        pl.kernel,
        mesh=mesh,
        out_type=jax.ShapeDtypeStruct((BATCH, EMBED), jnp.float32),
        scratch_types=[
            pltpu.VMEM((b_per_w,), jnp.int32),
            pltpu.VMEM((b_per_w, EMBED), jnp.float32),
            pltpu.SemaphoreType.DMA,
        ],
    )
    def gather(table_hbm, idx_hbm, out_hbm, idx_v, rows_v, sem):
        wid = lax.axis_index("s") * info.num_cores + lax.axis_index("c")
        base = wid * b_per_w
        pltpu.sync_copy(idx_hbm.at[pl.ds(base, b_per_w)], idx_v)
        pltpu.async_copy(table_hbm.at[idx_v], rows_v, sem).wait()
        pltpu.sync_copy(rows_v, out_hbm.at[pl.ds(base, b_per_w)])

    return gather


def _matmul_body(x_ref, w_ref, b_ref, o_hbm, o_tail, acc, sems):
    i = pl.program_id(0)
    n = pl.num_programs(0)
    slot = lax.rem(i, 2)
    col0 = i * V_TILE

    def copies(s, dst, col):
        return [
            pltpu.make_async_copy(
                acc.at[s, pl.ds(q * ROWS_Q, ROWS_Q), :],
                dst.at[pl.ds(q * ROWS_Q, ROWS_Q), pl.ds(col, V_TILE)],
                sems.at[s],
            )
            for q in range(NQ)
        ]

    @pl.when(i >= 2)
    def _():
        for d in copies(slot, o_hbm, 0):
            d.wait()

    val = lax.dot_general(
        x_ref[...], w_ref[...],
        (((1,), (1,)), ((), ())),
        preferred_element_type=jnp.float32,
    ) + b_ref[...]
    acc[pl.ds(slot, 1)] = val[None]

    @pl.when(i < n - 1)
    def _():
        for d in copies(slot, o_hbm, col0):
            d.start()

    @pl.when(i == n - 1)
    def _():
        for d in copies(slot, o_tail, 0):
            d.start()
        other = lax.rem(i + 1, 2)
        for d in copies(other, o_hbm, 0):
            d.wait()
        for d in copies(slot, o_hbm, 0):
            d.wait()


def _tc_logits(x, fc_w, fc_b2d):
    return pl.pallas_call(
        _matmul_body,
        grid=(GRID_V,),
        in_specs=[
            pl.BlockSpec((BATCH, EMBED), lambda i: (0, 0)),
            pl.BlockSpec((V_TILE, EMBED), lambda i: (i, 0)),
            pl.BlockSpec((1, V_TILE), lambda i: (0, i)),
        ],
        out_specs=[
            pl.BlockSpec(memory_space=pl.ANY),
            pl.BlockSpec(memory_space=pl.ANY),
        ],
        out_shape=[
            jax.ShapeDtypeStruct((BATCH, VOCAB), jnp.float32),
            jax.ShapeDtypeStruct((BATCH, V_TILE), jnp.float32),
        ],
        scratch_shapes=[
            pltpu.VMEM((2, BATCH, V_TILE), jnp.float32),
            pltpu.SemaphoreType.DMA((2,)),
        ],
        compiler_params=pltpu.CompilerParams(
            dimension_semantics=("arbitrary",),
        ),
    )(x, fc_w, fc_b2d)


def kernel(target_word, emb_table, fc_w, fc_b):
    x = _make_sc_gather()(emb_table, target_word.astype(jnp.int32))
    main, tail = _tc_logits(x, fc_w, fc_b.reshape(1, VOCAB))
    return lax.dynamic_update_slice(
        main, tail[:, :V_TAIL], (0, (GRID_V - 1) * V_TILE))
